# baseline (speedup 1.0000x reference)
import functools

import jax
import jax.numpy as jnp
from jax import lax
from jax.experimental import pallas as pl
from jax.experimental.pallas import tpu as pltpu
from jax.experimental.pallas import tpu_sc as plsc

N = 10000
E = 320000
D = 128
G = 64
NPAD = 10240
NT = 32
CW = 128
NCHUNK = 80
SEG = 40
CH0 = 40
CH1 = 120
TCHUNK = 16 * (CH0 + CH1)
EPAD = TCHUNK * CW
RPT = NPAD // 16


@functools.cache
def _mesh():
    return plsc.VectorSubcoreMesh(
        core_axis_name="c", subcore_axis_name="s",
        num_cores=2, num_subcores=16)


def _sc_deg_body(col_hbm, out_hbm, colv, onesb, dsh):
    cid = lax.axis_index("c")
    sid = lax.axis_index("s")
    t = cid * 16 + sid

    def zfill(i, _):
        onesb[i, :] = jnp.zeros((16,), jnp.float32)
        return 0
    lax.fori_loop(0, CW, zfill, 0)

    def zcopy(k, _):
        pltpu.sync_copy(onesb, dsh.at[pl.ds(sid * RPT + k * CW, CW)])
        return 0
    lax.fori_loop(0, RPT // CW, zcopy, 0)

    def ofill(i, _):
        onesb[i, :] = jnp.ones((16,), jnp.float32)
        return 0
    lax.fori_loop(0, CW, ofill, 0)

    pltpu.sync_copy(col_hbm.at[pl.ds(t * NCHUNK, NCHUNK)], colv)
    plsc.subcore_barrier()

    def scat(j, _):
        pltpu.sync_copy(onesb, dsh.at[colv.at[j]], add=True)
        return 0
    lax.fori_loop(0, NCHUNK, scat, 0)
    plsc.subcore_barrier()

    pltpu.sync_copy(dsh.at[pl.ds(sid * RPT, RPT)],
                    out_hbm.at[cid, pl.ds(sid * RPT, RPT)])


@functools.cache
def _sc_deg():
    return pl.kernel(
        _sc_deg_body,
        out_type=jax.ShapeDtypeStruct((2, NPAD, 16), jnp.float32),
        mesh=_mesh(),
        scratch_types=[
            pltpu.VMEM((NCHUNK, CW), jnp.int32),
            pltpu.VMEM((CW, 16), jnp.float32),
            pltpu.VMEM_SHARED((NPAD, 16), jnp.float32),
        ],
    )


def _sc_agg_body(g_hbm, row_hbm, col_hbm, out_hbm,
                 rowv, colv, bufa, bufb, ssh, sema, semb):
    cid = lax.axis_index("c")
    sid = lax.axis_index("s")
    t = cid * 16 + sid

    def zf(i, _):
        bufa[i // 8, pl.ds((i % 8) * 16, 16)] = jnp.zeros((16,), jnp.float32)
        return 0
    lax.fori_loop(0, CW * 8, zf, 0)

    def zc(k, _):
        pltpu.sync_copy(bufa, ssh.at[pl.ds(sid * RPT + k * CW, CW)])
        return 0
    lax.fori_loop(0, RPT // CW, zc, 0)
    plsc.subcore_barrier()

    nseg = lax.select(cid == 0, CH0 // SEG, CH1 // SEG)
    base = lax.select(cid == 0, sid * CH0, 16 * CH0 + sid * CH1)
    for seg in range(CH1 // SEG):

        @pl.when(seg < nseg)
        def _():
            start = base + seg * SEG
            pltpu.sync_copy(row_hbm.at[pl.ds(start, SEG)], rowv)
            pltpu.sync_copy(col_hbm.at[pl.ds(start, SEG)], colv)

            def body(j2, _):
                j = j2 * 2
                ha = pltpu.async_copy(g_hbm.at[rowv.at[j]], bufa, sema)
                ha.wait()
                hb = pltpu.async_copy(g_hbm.at[rowv.at[j + 1]], bufb, semb)
                pltpu.sync_copy(bufa, ssh.at[colv.at[j]], add=True)
                hb.wait()
                pltpu.sync_copy(bufb, ssh.at[colv.at[j + 1]], add=True)
                return 0
            lax.fori_loop(0, SEG // 2, body, 0)
    plsc.subcore_barrier()

    pltpu.sync_copy(ssh.at[pl.ds(sid * RPT, RPT)],
                    out_hbm.at[cid, pl.ds(sid * RPT, RPT)])


@functools.cache
def _sc_agg():
    return pl.kernel(
        _sc_agg_body,
        out_type=jax.ShapeDtypeStruct((2, NPAD, D), jnp.float32),
        mesh=_mesh(),
        scratch_types=[
            pltpu.VMEM((SEG, CW), jnp.int32),
            pltpu.VMEM((SEG, CW), jnp.int32),
            pltpu.VMEM((CW, D), jnp.float32),
            pltpu.VMEM((CW, D), jnp.float32),
            pltpu.VMEM_SHARED((NPAD, D), jnp.float32),
            pltpu.SemaphoreType.DMA,
            pltpu.SemaphoreType.DMA,
        ],
    )


_BLK = 1024
_GRID = NPAD // _BLK


def _tc1_body(deg_ref, x_ref, w_ref, g_ref, dv_ref):
    deg = deg_ref[0, :, 0:1] + deg_ref[1, :, 0:1] + 1.0
    dinv = lax.rsqrt(deg)
    h = jnp.dot(x_ref[...], w_ref[...], preferred_element_type=jnp.float32)
    g_ref[...] = h * dinv
    dv_ref[...] = jnp.broadcast_to(dinv, (_BLK, D))


_tc1 = pl.pallas_call(
    _tc1_body,
    grid=(_GRID,),
    in_specs=[
        pl.BlockSpec((2, _BLK, 16), lambda i: (0, i, 0)),
        pl.BlockSpec((_BLK, D), lambda i: (i, 0)),
        pl.BlockSpec((D, D), lambda i: (0, 0)),
    ],
    out_specs=[
        pl.BlockSpec((_BLK, D), lambda i: (i, 0)),
        pl.BlockSpec((_BLK, D), lambda i: (i, 0)),
    ],
    out_shape=[
        jax.ShapeDtypeStruct((NPAD, D), jnp.float32),
        jax.ShapeDtypeStruct((NPAD, D), jnp.float32),
    ],
)


def _tc2_body(s_ref, g_ref, dv_ref, b_ref, w_ref, out_ref):
    s = s_ref[0] + s_ref[1] + g_ref[...]
    z = jnp.maximum(s * dv_ref[...] + b_ref[...], 0.0)
    h = jnp.dot(z, w_ref[...], preferred_element_type=jnp.float32)
    out_ref[...] = h * dv_ref[...]


_tc2 = pl.pallas_call(
    _tc2_body,
    grid=(_GRID,),
    in_specs=[
        pl.BlockSpec((2, _BLK, D), lambda i: (0, i, 0)),
        pl.BlockSpec((_BLK, D), lambda i: (i, 0)),
        pl.BlockSpec((_BLK, D), lambda i: (i, 0)),
        pl.BlockSpec((1, D), lambda i: (0, 0)),
        pl.BlockSpec((D, D), lambda i: (0, 0)),
    ],
    out_specs=pl.BlockSpec((_BLK, D), lambda i: (i, 0)),
    out_shape=jax.ShapeDtypeStruct((NPAD, D), jnp.float32),
)


def _tc3_body(s_ref, g_ref, dv_ref, b_ref, bt_ref, wl_ref, bl_ref,
              out_ref, acc, cacc):
    i = pl.program_id(0)
    z = jnp.maximum((s_ref[0] + s_ref[1] + g_ref[...]) * dv_ref[...]
                    + b_ref[...], 0.0)
    gids = lax.broadcasted_iota(jnp.int32, (1, G), 1)
    mask = (bt_ref[...] == gids).astype(jnp.float32)
    psum = lax.dot_general(mask, z, (((0,), (0,)), ((), ())),
                           preferred_element_type=jnp.float32)
    ones = jnp.ones((_BLK, 1), jnp.float32)
    csum = lax.dot_general(mask, ones, (((0,), (0,)), ((), ())),
                           preferred_element_type=jnp.float32)

    @pl.when(i == 0)
    def _():
        acc[...] = psum
        cacc[...] = csum

    @pl.when(i > 0)
    def _():
        acc[...] += psum
        cacc[...] += csum

    @pl.when(i == pl.num_programs(0) - 1)
    def _():
        pooled = acc[...] / jnp.maximum(cacc[...], 1.0)
        out_ref[...] = jnp.dot(pooled, wl_ref[...],
                               preferred_element_type=jnp.float32) + bl_ref[...]


_tc3 = pl.pallas_call(
    _tc3_body,
    grid=(_GRID,),
    in_specs=[
        pl.BlockSpec((2, _BLK, D), lambda i: (0, i, 0)),
        pl.BlockSpec((_BLK, D), lambda i: (i, 0)),
        pl.BlockSpec((_BLK, D), lambda i: (i, 0)),
        pl.BlockSpec((1, D), lambda i: (0, 0)),
        pl.BlockSpec((_BLK, 1), lambda i: (i, 0)),
        pl.BlockSpec((D, 16), lambda i: (0, 0)),
        pl.BlockSpec((1, 16), lambda i: (0, 0)),
    ],
    out_specs=pl.BlockSpec((G, 16), lambda i: (0, 0)),
    out_shape=jax.ShapeDtypeStruct((G, 16), jnp.float32),
    scratch_shapes=[
        pltpu.VMEM((G, D), jnp.float32),
        pltpu.VMEM((G, 1), jnp.float32),
    ],
)


def kernel(x, edge_index, batch, W1, b1, W2, b2, Wl, bl):
    row = edge_index[0]
    col = edge_index[1]
    pad = EPAD - E
    rowp = jnp.concatenate(
        [row, jnp.zeros((pad,), jnp.int32)]).reshape(TCHUNK, CW)
    colp = jnp.concatenate(
        [col, jnp.full((pad,), N, jnp.int32)]).reshape(TCHUNK, CW)
    xp = jnp.pad(x, ((0, NPAD - N), (0, 0)))
    b2d = jnp.pad(batch, (0, NPAD - N), constant_values=G)[:, None]

    deg = _sc_deg()(colp)
    g1, dinvb = _tc1(deg, xp, W1)
    S1 = _sc_agg()(g1, rowp, colp)
    g2 = _tc2(S1, g1, dinvb, b1.reshape(1, D), W2)
    S2 = _sc_agg()(g2, rowp, colp)
    out = _tc3(S2, g2, dinvb, b2.reshape(1, D), b2d, Wl, bl.reshape(1, 16))
    return out


# speedup vs baseline: 1.0002x; 1.0002x over previous
import functools

import jax
import jax.numpy as jnp
from jax import lax
from jax.experimental import pallas as pl
from jax.experimental.pallas import tpu as pltpu
from jax.experimental.pallas import tpu_sc as plsc

N = 10000
E = 320000
D = 128
G = 64
NPAD = 10240
NT = 32
CW = 128
NCHUNK = 80
SEG = 40
CH0 = 40
CH1 = 120
TCHUNK = 16 * (CH0 + CH1)
EPAD = TCHUNK * CW
RPT = NPAD // 16


@functools.cache
def _mesh():
    return plsc.VectorSubcoreMesh(
        core_axis_name="c", subcore_axis_name="s",
        num_cores=2, num_subcores=16)


def _sc_deg_body(col_hbm, out_hbm, colv, onesb, dsh):
    cid = lax.axis_index("c")
    sid = lax.axis_index("s")
    t = cid * 16 + sid

    def zfill(i, _):
        onesb[i, :] = jnp.zeros((16,), jnp.float32)
        return 0
    lax.fori_loop(0, CW, zfill, 0)

    def zcopy(k, _):
        pltpu.sync_copy(onesb, dsh.at[pl.ds(sid * RPT + k * CW, CW)])
        return 0
    lax.fori_loop(0, RPT // CW, zcopy, 0)

    def ofill(i, _):
        onesb[i, :] = jnp.ones((16,), jnp.float32)
        return 0
    lax.fori_loop(0, CW, ofill, 0)

    pltpu.sync_copy(col_hbm.at[pl.ds(t * NCHUNK, NCHUNK)], colv)
    plsc.subcore_barrier()

    def scat(j, _):
        pltpu.sync_copy(onesb, dsh.at[colv.at[j]], add=True)
        return 0
    lax.fori_loop(0, NCHUNK, scat, 0)
    plsc.subcore_barrier()

    pltpu.sync_copy(dsh.at[pl.ds(sid * RPT, RPT)],
                    out_hbm.at[cid, pl.ds(sid * RPT, RPT)])


@functools.cache
def _sc_deg():
    return pl.kernel(
        _sc_deg_body,
        out_type=jax.ShapeDtypeStruct((2, NPAD, 16), jnp.float32),
        mesh=_mesh(),
        scratch_types=[
            pltpu.VMEM((NCHUNK, CW), jnp.int32),
            pltpu.VMEM((CW, 16), jnp.float32),
            pltpu.VMEM_SHARED((NPAD, 16), jnp.float32),
        ],
    )


def _sc_agg_body(g_hbm, row_hbm, col_hbm, out_hbm,
                 rowv, colv, bufa, bufb, ssh, sema, semb):
    cid = lax.axis_index("c")
    sid = lax.axis_index("s")
    t = cid * 16 + sid

    def zf(i, _):
        bufa[i // 8, pl.ds((i % 8) * 16, 16)] = jnp.zeros((16,), jnp.float32)
        return 0
    lax.fori_loop(0, CW * 8, zf, 0)

    def zc(k, _):
        pltpu.sync_copy(bufa, ssh.at[pl.ds(sid * RPT + k * CW, CW)])
        return 0
    lax.fori_loop(0, RPT // CW, zc, 0)
    plsc.subcore_barrier()

    nseg = lax.select(cid == 0, CH0 // SEG, CH1 // SEG)
    base = lax.select(cid == 0, sid * CH0, 16 * CH0 + sid * CH1)

    def seg_body(seg, _):
        start = base + seg * SEG
        pltpu.sync_copy(row_hbm.at[pl.ds(start, SEG)], rowv)
        pltpu.sync_copy(col_hbm.at[pl.ds(start, SEG)], colv)

        def body(j2, _):
            j = j2 * 2
            ha = pltpu.async_copy(g_hbm.at[rowv.at[j]], bufa, sema)
            ha.wait()
            hb = pltpu.async_copy(g_hbm.at[rowv.at[j + 1]], bufb, semb)
            pltpu.sync_copy(bufa, ssh.at[colv.at[j]], add=True)
            hb.wait()
            pltpu.sync_copy(bufb, ssh.at[colv.at[j + 1]], add=True)
            return 0
        lax.fori_loop(0, SEG // 2, body, 0)
        return 0
    lax.fori_loop(0, nseg, seg_body, 0)
    plsc.subcore_barrier()

    pltpu.sync_copy(ssh.at[pl.ds(sid * RPT, RPT)],
                    out_hbm.at[cid, pl.ds(sid * RPT, RPT)])


@functools.cache
def _sc_agg():
    return pl.kernel(
        _sc_agg_body,
        out_type=jax.ShapeDtypeStruct((2, NPAD, D), jnp.float32),
        mesh=_mesh(),
        scratch_types=[
            pltpu.VMEM((SEG, CW), jnp.int32),
            pltpu.VMEM((SEG, CW), jnp.int32),
            pltpu.VMEM((CW, D), jnp.float32),
            pltpu.VMEM((CW, D), jnp.float32),
            pltpu.VMEM_SHARED((NPAD, D), jnp.float32),
            pltpu.SemaphoreType.DMA,
            pltpu.SemaphoreType.DMA,
        ],
    )


_BLK = 1024
_GRID = NPAD // _BLK


def _tc1_body(deg_ref, x_ref, w_ref, g_ref, dv_ref):
    deg = deg_ref[0, :, 0:1] + deg_ref[1, :, 0:1] + 1.0
    dinv = lax.rsqrt(deg)
    h = jnp.dot(x_ref[...], w_ref[...], preferred_element_type=jnp.float32)
    g_ref[...] = h * dinv
    dv_ref[...] = jnp.broadcast_to(dinv, (_BLK, D))


_tc1 = pl.pallas_call(
    _tc1_body,
    grid=(_GRID,),
    in_specs=[
        pl.BlockSpec((2, _BLK, 16), lambda i: (0, i, 0)),
        pl.BlockSpec((_BLK, D), lambda i: (i, 0)),
        pl.BlockSpec((D, D), lambda i: (0, 0)),
    ],
    out_specs=[
        pl.BlockSpec((_BLK, D), lambda i: (i, 0)),
        pl.BlockSpec((_BLK, D), lambda i: (i, 0)),
    ],
    out_shape=[
        jax.ShapeDtypeStruct((NPAD, D), jnp.float32),
        jax.ShapeDtypeStruct((NPAD, D), jnp.float32),
    ],
)


def _tc2_body(s_ref, g_ref, dv_ref, b_ref, w_ref, out_ref):
    s = s_ref[0] + s_ref[1] + g_ref[...]
    z = jnp.maximum(s * dv_ref[...] + b_ref[...], 0.0)
    h = jnp.dot(z, w_ref[...], preferred_element_type=jnp.float32)
    out_ref[...] = h * dv_ref[...]


_tc2 = pl.pallas_call(
    _tc2_body,
    grid=(_GRID,),
    in_specs=[
        pl.BlockSpec((2, _BLK, D), lambda i: (0, i, 0)),
        pl.BlockSpec((_BLK, D), lambda i: (i, 0)),
        pl.BlockSpec((_BLK, D), lambda i: (i, 0)),
        pl.BlockSpec((1, D), lambda i: (0, 0)),
        pl.BlockSpec((D, D), lambda i: (0, 0)),
    ],
    out_specs=pl.BlockSpec((_BLK, D), lambda i: (i, 0)),
    out_shape=jax.ShapeDtypeStruct((NPAD, D), jnp.float32),
)


def _tc3_body(s_ref, g_ref, dv_ref, b_ref, bt_ref, wl_ref, bl_ref,
              out_ref, acc, cacc):
    i = pl.program_id(0)
    z = jnp.maximum((s_ref[0] + s_ref[1] + g_ref[...]) * dv_ref[...]
                    + b_ref[...], 0.0)
    gids = lax.broadcasted_iota(jnp.int32, (1, G), 1)
    mask = (bt_ref[...] == gids).astype(jnp.float32)
    psum = lax.dot_general(mask, z, (((0,), (0,)), ((), ())),
                           preferred_element_type=jnp.float32)
    ones = jnp.ones((_BLK, 1), jnp.float32)
    csum = lax.dot_general(mask, ones, (((0,), (0,)), ((), ())),
                           preferred_element_type=jnp.float32)

    @pl.when(i == 0)
    def _():
        acc[...] = psum
        cacc[...] = csum

    @pl.when(i > 0)
    def _():
        acc[...] += psum
        cacc[...] += csum

    @pl.when(i == pl.num_programs(0) - 1)
    def _():
        pooled = acc[...] / jnp.maximum(cacc[...], 1.0)
        out_ref[...] = jnp.dot(pooled, wl_ref[...],
                               preferred_element_type=jnp.float32) + bl_ref[...]


_tc3 = pl.pallas_call(
    _tc3_body,
    grid=(_GRID,),
    in_specs=[
        pl.BlockSpec((2, _BLK, D), lambda i: (0, i, 0)),
        pl.BlockSpec((_BLK, D), lambda i: (i, 0)),
        pl.BlockSpec((_BLK, D), lambda i: (i, 0)),
        pl.BlockSpec((1, D), lambda i: (0, 0)),
        pl.BlockSpec((_BLK, 1), lambda i: (i, 0)),
        pl.BlockSpec((D, 16), lambda i: (0, 0)),
        pl.BlockSpec((1, 16), lambda i: (0, 0)),
    ],
    out_specs=pl.BlockSpec((G, 16), lambda i: (0, 0)),
    out_shape=jax.ShapeDtypeStruct((G, 16), jnp.float32),
    scratch_shapes=[
        pltpu.VMEM((G, D), jnp.float32),
        pltpu.VMEM((G, 1), jnp.float32),
    ],
)


def kernel(x, edge_index, batch, W1, b1, W2, b2, Wl, bl):
    row = edge_index[0]
    col = edge_index[1]
    pad = EPAD - E
    rowp = jnp.concatenate(
        [row, jnp.zeros((pad,), jnp.int32)]).reshape(TCHUNK, CW)
    colp = jnp.concatenate(
        [col, jnp.full((pad,), N, jnp.int32)]).reshape(TCHUNK, CW)
    xp = jnp.pad(x, ((0, NPAD - N), (0, 0)))
    b2d = jnp.pad(batch, (0, NPAD - N), constant_values=G)[:, None]

    deg = _sc_deg()(colp)
    g1, dinvb = _tc1(deg, xp, W1)
    S1 = _sc_agg()(g1, rowp, colp)
    g2 = _tc2(S1, g1, dinvb, b1.reshape(1, D), W2)
    S2 = _sc_agg()(g2, rowp, colp)
    out = _tc3(S2, g2, dinvb, b2.reshape(1, D), b2d, Wl, bl.reshape(1, 16))
    return out


# speedup vs baseline: 1.2362x; 1.2359x over previous
import functools

import jax
import jax.numpy as jnp
from jax import lax
from jax.experimental import pallas as pl
from jax.experimental.pallas import tpu as pltpu
from jax.experimental.pallas import tpu_sc as plsc

N = 10000
E = 320000
D = 128
G = 64
NPAD = 10240
NT = 32
CW = 128
NCHUNK = 80
SEG = 40
CH0 = 120
CH1 = 40
TCHUNK = 16 * (CH0 + CH1)
EPAD = TCHUNK * CW
RPT = NPAD // 16


@functools.cache
def _mesh():
    return plsc.VectorSubcoreMesh(
        core_axis_name="c", subcore_axis_name="s",
        num_cores=2, num_subcores=16)


def _sc_deg_body(col_hbm, out_hbm, colv, onesb, dsh):
    cid = lax.axis_index("c")
    sid = lax.axis_index("s")
    t = cid * 16 + sid

    def zfill(i, _):
        onesb[i, :] = jnp.zeros((16,), jnp.float32)
        return 0
    lax.fori_loop(0, CW, zfill, 0)

    def zcopy(k, _):
        pltpu.sync_copy(onesb, dsh.at[pl.ds(sid * RPT + k * CW, CW)])
        return 0
    lax.fori_loop(0, RPT // CW, zcopy, 0)

    def ofill(i, _):
        onesb[i, :] = jnp.ones((16,), jnp.float32)
        return 0
    lax.fori_loop(0, CW, ofill, 0)

    pltpu.sync_copy(col_hbm.at[pl.ds(t * NCHUNK, NCHUNK)], colv)
    plsc.subcore_barrier()

    def scat(j, _):
        pltpu.sync_copy(onesb, dsh.at[colv.at[j]], add=True)
        return 0
    lax.fori_loop(0, NCHUNK, scat, 0)
    plsc.subcore_barrier()

    pltpu.sync_copy(dsh.at[pl.ds(sid * RPT, RPT)],
                    out_hbm.at[cid, pl.ds(sid * RPT, RPT)])


@functools.cache
def _sc_deg():
    return pl.kernel(
        _sc_deg_body,
        out_type=jax.ShapeDtypeStruct((2, NPAD, 16), jnp.float32),
        mesh=_mesh(),
        scratch_types=[
            pltpu.VMEM((NCHUNK, CW), jnp.int32),
            pltpu.VMEM((CW, 16), jnp.float32),
            pltpu.VMEM_SHARED((NPAD, 16), jnp.float32),
        ],
    )


def _sc_agg_body(g_hbm, row_hbm, col_hbm, out_hbm,
                 rowv, colv, bufa, bufb, ssh, sema, semb):
    cid = lax.axis_index("c")
    sid = lax.axis_index("s")
    t = cid * 16 + sid

    def zf(i, _):
        bufa[i // 8, pl.ds((i % 8) * 16, 16)] = jnp.zeros((16,), jnp.float32)
        return 0
    lax.fori_loop(0, CW * 8, zf, 0)

    def zc(k, _):
        pltpu.sync_copy(bufa, ssh.at[pl.ds(sid * RPT + k * CW, CW)])
        return 0
    lax.fori_loop(0, RPT // CW, zc, 0)
    plsc.subcore_barrier()

    nseg = lax.select(cid == 0, CH0 // SEG, CH1 // SEG)
    base = lax.select(cid == 0, sid * CH0, 16 * CH0 + sid * CH1)

    def seg_body(seg, _):
        start = base + seg * SEG
        pltpu.sync_copy(row_hbm.at[pl.ds(start, SEG)], rowv)
        pltpu.sync_copy(col_hbm.at[pl.ds(start, SEG)], colv)

        def body(j2, _):
            j = j2 * 2
            ha = pltpu.async_copy(g_hbm.at[rowv.at[j]], bufa, sema)
            ha.wait()
            hb = pltpu.async_copy(g_hbm.at[rowv.at[j + 1]], bufb, semb)
            pltpu.sync_copy(bufa, ssh.at[colv.at[j]], add=True)
            hb.wait()
            pltpu.sync_copy(bufb, ssh.at[colv.at[j + 1]], add=True)
            return 0
        lax.fori_loop(0, SEG // 2, body, 0)
        return 0
    lax.fori_loop(0, nseg, seg_body, 0)
    plsc.subcore_barrier()

    pltpu.sync_copy(ssh.at[pl.ds(sid * RPT, RPT)],
                    out_hbm.at[cid, pl.ds(sid * RPT, RPT)])


@functools.cache
def _sc_agg():
    return pl.kernel(
        _sc_agg_body,
        out_type=jax.ShapeDtypeStruct((2, NPAD, D), jnp.float32),
        mesh=_mesh(),
        scratch_types=[
            pltpu.VMEM((SEG, CW), jnp.int32),
            pltpu.VMEM((SEG, CW), jnp.int32),
            pltpu.VMEM((CW, D), jnp.float32),
            pltpu.VMEM((CW, D), jnp.float32),
            pltpu.VMEM_SHARED((NPAD, D), jnp.float32),
            pltpu.SemaphoreType.DMA,
            pltpu.SemaphoreType.DMA,
        ],
    )


_BLK = 1024
_GRID = NPAD // _BLK


def _tc1_body(deg_ref, x_ref, w_ref, g_ref, dv_ref):
    deg = deg_ref[0, :, 0:1] + deg_ref[1, :, 0:1] + 1.0
    dinv = lax.rsqrt(deg)
    h = jnp.dot(x_ref[...], w_ref[...], preferred_element_type=jnp.float32)
    g_ref[...] = h * dinv
    dv_ref[...] = jnp.broadcast_to(dinv, (_BLK, D))


_tc1 = pl.pallas_call(
    _tc1_body,
    grid=(_GRID,),
    in_specs=[
        pl.BlockSpec((2, _BLK, 16), lambda i: (0, i, 0)),
        pl.BlockSpec((_BLK, D), lambda i: (i, 0)),
        pl.BlockSpec((D, D), lambda i: (0, 0)),
    ],
    out_specs=[
        pl.BlockSpec((_BLK, D), lambda i: (i, 0)),
        pl.BlockSpec((_BLK, D), lambda i: (i, 0)),
    ],
    out_shape=[
        jax.ShapeDtypeStruct((NPAD, D), jnp.float32),
        jax.ShapeDtypeStruct((NPAD, D), jnp.float32),
    ],
)


def _tc2_body(s_ref, g_ref, dv_ref, b_ref, w_ref, out_ref):
    s = s_ref[0] + s_ref[1] + g_ref[...]
    z = jnp.maximum(s * dv_ref[...] + b_ref[...], 0.0)
    h = jnp.dot(z, w_ref[...], preferred_element_type=jnp.float32)
    out_ref[...] = h * dv_ref[...]


_tc2 = pl.pallas_call(
    _tc2_body,
    grid=(_GRID,),
    in_specs=[
        pl.BlockSpec((2, _BLK, D), lambda i: (0, i, 0)),
        pl.BlockSpec((_BLK, D), lambda i: (i, 0)),
        pl.BlockSpec((_BLK, D), lambda i: (i, 0)),
        pl.BlockSpec((1, D), lambda i: (0, 0)),
        pl.BlockSpec((D, D), lambda i: (0, 0)),
    ],
    out_specs=pl.BlockSpec((_BLK, D), lambda i: (i, 0)),
    out_shape=jax.ShapeDtypeStruct((NPAD, D), jnp.float32),
)


def _tc3_body(s_ref, g_ref, dv_ref, b_ref, bt_ref, wl_ref, bl_ref,
              out_ref, acc, cacc):
    i = pl.program_id(0)
    z = jnp.maximum((s_ref[0] + s_ref[1] + g_ref[...]) * dv_ref[...]
                    + b_ref[...], 0.0)
    gids = lax.broadcasted_iota(jnp.int32, (1, G), 1)
    mask = (bt_ref[...] == gids).astype(jnp.float32)
    psum = lax.dot_general(mask, z, (((0,), (0,)), ((), ())),
                           preferred_element_type=jnp.float32)
    ones = jnp.ones((_BLK, 1), jnp.float32)
    csum = lax.dot_general(mask, ones, (((0,), (0,)), ((), ())),
                           preferred_element_type=jnp.float32)

    @pl.when(i == 0)
    def _():
        acc[...] = psum
        cacc[...] = csum

    @pl.when(i > 0)
    def _():
        acc[...] += psum
        cacc[...] += csum

    @pl.when(i == pl.num_programs(0) - 1)
    def _():
        pooled = acc[...] / jnp.maximum(cacc[...], 1.0)
        out_ref[...] = jnp.dot(pooled, wl_ref[...],
                               preferred_element_type=jnp.float32) + bl_ref[...]


_tc3 = pl.pallas_call(
    _tc3_body,
    grid=(_GRID,),
    in_specs=[
        pl.BlockSpec((2, _BLK, D), lambda i: (0, i, 0)),
        pl.BlockSpec((_BLK, D), lambda i: (i, 0)),
        pl.BlockSpec((_BLK, D), lambda i: (i, 0)),
        pl.BlockSpec((1, D), lambda i: (0, 0)),
        pl.BlockSpec((_BLK, 1), lambda i: (i, 0)),
        pl.BlockSpec((D, 16), lambda i: (0, 0)),
        pl.BlockSpec((1, 16), lambda i: (0, 0)),
    ],
    out_specs=pl.BlockSpec((G, 16), lambda i: (0, 0)),
    out_shape=jax.ShapeDtypeStruct((G, 16), jnp.float32),
    scratch_shapes=[
        pltpu.VMEM((G, D), jnp.float32),
        pltpu.VMEM((G, 1), jnp.float32),
    ],
)


def kernel(x, edge_index, batch, W1, b1, W2, b2, Wl, bl):
    row = edge_index[0]
    col = edge_index[1]
    pad = EPAD - E
    rowp = jnp.concatenate(
        [row, jnp.zeros((pad,), jnp.int32)]).reshape(TCHUNK, CW)
    colp = jnp.concatenate(
        [col, jnp.full((pad,), N, jnp.int32)]).reshape(TCHUNK, CW)
    xp = jnp.pad(x, ((0, NPAD - N), (0, 0)))
    b2d = jnp.pad(batch, (0, NPAD - N), constant_values=G)[:, None]

    deg = _sc_deg()(colp)
    g1, dinvb = _tc1(deg, xp, W1)
    S1 = _sc_agg()(g1, rowp, colp)
    g2 = _tc2(S1, g1, dinvb, b1.reshape(1, D), W2)
    S2 = _sc_agg()(g2, rowp, colp)
    out = _tc3(S2, g2, dinvb, b2.reshape(1, D), b2d, Wl, bl.reshape(1, 16))
    return out
